# baseline (speedup 1.0000x reference)
import jax
import jax.numpy as jnp
from jax import lax
from jax.experimental import pallas as pl
from jax.experimental.pallas import tpu as pltpu
from jax.experimental.pallas import tpu_sc as plsc

B = 16384
H = 200
D = 64
NC = 2
NS = 16
NW = NC * NS
RW = B // NW
CH = 128
NCH = RW // CH

K = 8
DEP = 7
GD = K - DEP


def _sc_body(xt_hbm, emb_hbm, out_hbm, *refs):
    idx = list(refs[0:K])
    acc_v = refs[K]
    gsem = list(refs[K + 1:2 * K + 1])
    isem = list(refs[2 * K + 1:3 * K + 1])

    c = lax.axis_index("c")
    s = lax.axis_index("s")
    wid = c * NS + s

    def fire_idx(h, j):
        pltpu.async_copy(xt_hbm.at[h, wid], idx[j], isem[j])

    def wait_idx(j):
        pltpu.make_async_copy(xt_hbm.at[0, 0], idx[j], isem[j]).wait()

    def fire_gathers(j, add):
        for k in range(NCH):
            pltpu.async_copy(
                emb_hbm.at[idx[j].at[pl.ds(k * CH, CH)]],
                acc_v.at[pl.ds(k * CH, CH)],
                gsem[j],
                add=add,
            )

    def drain_gathers(j):
        pltpu.make_async_copy(emb_hbm.at[pl.ds(0, RW)], acc_v, gsem[j]).wait()

    def step(h, jj, add=True, drain=True, refill=True):
        wait_idx(jj)
        if drain:
            drain_gathers((jj + DEP) % K)
        fire_gathers(jj, add)
        if refill:
            fire_idx(h + DEP, (jj + DEP) % K)

    for j in range(K):
        fire_idx(j, j)

    step(0, 0, add=False, drain=False, refill=False)
    for h in range(1, GD):
        step(h, h, drain=False, refill=False)
    for h in range(GD, K):
        step(h, h)

    def group(g, carry):
        h8 = K * g
        for jj in range(K):
            step(h8 + jj, jj)
        return carry

    lax.fori_loop(1, H // K - 1, group, 0)

    for h in range(H - K, H):
        step(h, h % K, refill=(h + DEP < H))
    for h in range(H - GD, H):
        drain_gathers(h % K)

    pltpu.sync_copy(acc_v, out_hbm.at[pl.ds(wid * RW, RW)])


@jax.jit
def _sc_sum(xt, emb):
    mesh = plsc.VectorSubcoreMesh(core_axis_name="c", subcore_axis_name="s")
    fn = pl.kernel(
        _sc_body,
        out_type=jax.ShapeDtypeStruct((B, D), jnp.float32),
        mesh=mesh,
        scratch_types=(
            [pltpu.VMEM((RW,), jnp.int32)] * K
            + [pltpu.VMEM((RW, D), jnp.float32)]
            + [pltpu.SemaphoreType.DMA] * (2 * K)
        ),
        compiler_params=pltpu.CompilerParams(use_tc_tiling_on_sc=False),
    )
    return fn(xt, emb)


BLK = 512


def _tc_body(x_ref, sum_ref, w_ref, b_ref, o_ref):
    cnt = jnp.sum((x_ref[...] != 0).astype(jnp.float32), axis=1, keepdims=True)
    mean = sum_ref[...] / (cnt + 1e-6)
    o_ref[...] = (
        lax.dot_general(
            mean, w_ref[...], (((1,), (1,)), ((), ())),
            preferred_element_type=jnp.float32,
        )
        + b_ref[...]
    )


@jax.jit
def _tc_finish(x, summed, W, b2):
    return pl.pallas_call(
        _tc_body,
        grid=(B // BLK,),
        in_specs=[
            pl.BlockSpec((BLK, H), lambda i: (i, 0)),
            pl.BlockSpec((BLK, D), lambda i: (i, 0)),
            pl.BlockSpec((D, D), lambda i: (0, 0)),
            pl.BlockSpec((1, D), lambda i: (0, 0)),
        ],
        out_specs=pl.BlockSpec((BLK, D), lambda i: (i, 0)),
        out_shape=jax.ShapeDtypeStruct((B, D), jnp.float32),
    )(x, summed, W, b2)


def kernel(x, lengths, emb, W, b):
    x = jnp.asarray(x, jnp.int32)
    xt = x.T.reshape(H, NW, RW)
    summed = _sc_sum(xt, emb)
    return _tc_finish(x, summed, W, b.reshape(1, D))


# speedup vs baseline: 1.0484x; 1.0484x over previous
import jax
import jax.numpy as jnp
from jax import lax
from jax.experimental import pallas as pl
from jax.experimental.pallas import tpu as pltpu
from jax.experimental.pallas import tpu_sc as plsc

B = 16384
H = 200
D = 64
NC = 2
NS = 16
NW = NC * NS
RW = B // NW
CH = 128
NCH = RW // CH

K = 8
DEP = 6
GD = K - DEP


def _sc_body(xt_hbm, emb_hbm, out_hbm, *refs):
    idx = list(refs[0:K])
    acc = list(refs[K:K + GD])
    gsem = list(refs[K + GD:2 * K + GD])
    isem = list(refs[2 * K + GD:3 * K + GD])

    c = lax.axis_index("c")
    s = lax.axis_index("s")
    wid = c * NS + s

    def fire_idx(h, j):
        pltpu.async_copy(xt_hbm.at[h, wid], idx[j], isem[j])

    def wait_idx(j):
        pltpu.make_async_copy(xt_hbm.at[0, 0], idx[j], isem[j]).wait()

    def fire_gathers(j, add, p):
        for k in range(NCH):
            pltpu.async_copy(
                emb_hbm.at[idx[j].at[pl.ds(k * CH, CH)]],
                acc[p].at[pl.ds(k * CH, CH)],
                gsem[j],
                add=add,
            )

    def drain_gathers(j):
        pltpu.make_async_copy(
            emb_hbm.at[pl.ds(0, RW)], acc[j % GD], gsem[j]
        ).wait()

    def step(h, jj, add=True, drain=True, refill=True):
        wait_idx(jj)
        if drain:
            drain_gathers((jj + DEP) % K)
        fire_gathers(jj, add, jj % GD)
        if refill:
            fire_idx(h + DEP, (jj + DEP) % K)

    for j in range(K):
        fire_idx(j, j)

    step(0, 0, add=False, drain=False, refill=False)
    for h in range(1, GD):
        step(h, h, add=False, drain=False, refill=False)
    for h in range(GD, K):
        step(h, h)

    def group(g, carry):
        h8 = K * g
        for jj in range(K):
            step(h8 + jj, jj)
        return carry

    lax.fori_loop(1, H // K - 1, group, 0)

    for h in range(H - K, H):
        step(h, h % K, refill=(h + DEP < H))
    for h in range(H - GD, H):
        drain_gathers(h % K)

    def comb(b, carry):
        for k in range(D // 16):
            v = acc[0][b, pl.ds(k * 16, 16)]
            for p in range(1, GD):
                v = v + acc[p][b, pl.ds(k * 16, 16)]
            acc[0][b, pl.ds(k * 16, 16)] = v
        return carry

    lax.fori_loop(0, RW, comb, 0)

    pltpu.sync_copy(acc[0], out_hbm.at[pl.ds(wid * RW, RW)])


@jax.jit
def _sc_sum(xt, emb):
    mesh = plsc.VectorSubcoreMesh(core_axis_name="c", subcore_axis_name="s")
    fn = pl.kernel(
        _sc_body,
        out_type=jax.ShapeDtypeStruct((B, D), jnp.float32),
        mesh=mesh,
        scratch_types=(
            [pltpu.VMEM((RW,), jnp.int32)] * K
            + [pltpu.VMEM((RW, D), jnp.float32)] * GD
            + [pltpu.SemaphoreType.DMA] * (2 * K)
        ),
        compiler_params=pltpu.CompilerParams(use_tc_tiling_on_sc=False),
    )
    return fn(xt, emb)


BLK = 512


def _tc_body(x_ref, sum_ref, w_ref, b_ref, o_ref):
    cnt = jnp.sum((x_ref[...] != 0).astype(jnp.float32), axis=1, keepdims=True)
    mean = sum_ref[...] / (cnt + 1e-6)
    o_ref[...] = (
        lax.dot_general(
            mean, w_ref[...], (((1,), (1,)), ((), ())),
            preferred_element_type=jnp.float32,
        )
        + b_ref[...]
    )


@jax.jit
def _tc_finish(x, summed, W, b2):
    return pl.pallas_call(
        _tc_body,
        grid=(B // BLK,),
        in_specs=[
            pl.BlockSpec((BLK, H), lambda i: (i, 0)),
            pl.BlockSpec((BLK, D), lambda i: (i, 0)),
            pl.BlockSpec((D, D), lambda i: (0, 0)),
            pl.BlockSpec((1, D), lambda i: (0, 0)),
        ],
        out_specs=pl.BlockSpec((BLK, D), lambda i: (i, 0)),
        out_shape=jax.ShapeDtypeStruct((B, D), jnp.float32),
    )(x, summed, W, b2)


def kernel(x, lengths, emb, W, b):
    x = jnp.asarray(x, jnp.int32)
    xt = x.T.reshape(H, NW, RW)
    summed = _sc_sum(xt, emb)
    return _tc_finish(x, summed, W, b.reshape(1, D))


# speedup vs baseline: 1.1298x; 1.0777x over previous
import functools

import jax
import jax.numpy as jnp
from jax import lax
from jax.experimental import pallas as pl
from jax.experimental.pallas import tpu as pltpu
from jax.experimental.pallas import tpu_sc as plsc

B = 16384
H = 200
D = 64
NC = 2
NS = 16
NW = NC * NS
RW = B // NW
CH = 128
NCH = RW // CH


CH2 = H - CH


NSLOT = 6
GA = 3


def _sc_body(x_hbm, emb_hbm, out_hbm, *refs):
    idx = list(refs[0:NSLOT])
    rows = list(refs[NSLOT:2 * NSLOT])
    acc_v = refs[2 * NSLOT]
    gsem = list(refs[2 * NSLOT + 1:3 * NSLOT + 1])
    isem = list(refs[3 * NSLOT + 1:4 * NSLOT + 1])

    c = lax.axis_index("c")
    s = lax.axis_index("s")
    wid = c * NS + s
    base = wid * RW

    def fire_idx(b, j):
        pltpu.async_copy(x_hbm.at[base + b], idx[j], isem[j])

    def wait_idx(j):
        pltpu.make_async_copy(x_hbm.at[0], idx[j], isem[j]).wait()

    def fire_gathers(idx_ref, rows_ref, sem):
        pltpu.async_copy(
            emb_hbm.at[idx_ref.at[pl.ds(0, CH)]], rows_ref.at[pl.ds(0, CH)], sem
        )
        pltpu.async_copy(
            emb_hbm.at[idx_ref.at[pl.ds(CH, CH2)]],
            rows_ref.at[pl.ds(CH, CH2)],
            sem,
        )

    def wait_gathers(rows_ref, sem):
        pltpu.make_async_copy(emb_hbm.at[pl.ds(0, H)], rows_ref, sem).wait()

    z = jnp.zeros((16,), jnp.float32)

    def reduce_into(rows_ref, b):
        @plsc.parallel_loop(0, H // 2, unroll=4, carry=(z,) * 8)
        def _red(r, p):
            lo = [rows_ref[2 * r, pl.ds(k * 16, 16)] for k in range(4)]
            hi = [rows_ref[2 * r + 1, pl.ds(k * 16, 16)] for k in range(4)]
            return tuple(p[k] + lo[k] for k in range(4)) + tuple(
                p[4 + k] + hi[k] for k in range(4)
            )

        for k in range(4):
            acc_v[b, pl.ds(k * 16, 16)] = _red[k] + _red[4 + k]

    def stage(b, j, fire_g=True, fire_i=True):
        jg = (j + GA) % NSLOT
        if fire_g:
            wait_idx(jg)
            fire_gathers(idx[jg], rows[jg], gsem[jg])
        wait_gathers(rows[j], gsem[j])
        if fire_i:
            fire_idx(b + NSLOT, j)
        reduce_into(rows[j], b)

    for j in range(NSLOT):
        fire_idx(j, j)
    for j in range(GA):
        wait_idx(j)
        fire_gathers(idx[j], rows[j], gsem[j])

    NMAIN = (RW - NSLOT) // NSLOT * NSLOT

    def group(i, carry):
        b0 = NSLOT * i
        for j in range(NSLOT):
            stage(b0 + j, j)
        return carry

    lax.fori_loop(0, NMAIN // NSLOT, group, 0)

    for b in range(NMAIN, RW):
        stage(b, b % NSLOT, fire_g=(b + GA < RW), fire_i=(b + NSLOT < RW))

    pltpu.sync_copy(acc_v, out_hbm.at[pl.ds(base, RW)])


@jax.jit
def _sc_sum(x, emb):
    mesh = plsc.VectorSubcoreMesh(core_axis_name="c", subcore_axis_name="s")
    fn = pl.kernel(
        _sc_body,
        out_type=jax.ShapeDtypeStruct((B, D), jnp.float32),
        mesh=mesh,
        scratch_types=(
            [pltpu.VMEM((H,), jnp.int32)] * NSLOT
            + [pltpu.VMEM((H, D), jnp.float32)] * NSLOT
            + [pltpu.VMEM((RW, D), jnp.float32)]
            + [pltpu.SemaphoreType.DMA] * (2 * NSLOT)
        ),
        compiler_params=pltpu.CompilerParams(use_tc_tiling_on_sc=False),
    )
    return fn(x, emb)


BLK = 512


def _tc_body(x_ref, sum_ref, w_ref, b_ref, o_ref):
    cnt = jnp.sum((x_ref[...] != 0).astype(jnp.float32), axis=1, keepdims=True)
    mean = sum_ref[...] / (cnt + 1e-6)
    o_ref[...] = (
        lax.dot_general(
            mean, w_ref[...], (((1,), (1,)), ((), ())),
            preferred_element_type=jnp.float32,
        )
        + b_ref[...]
    )


@jax.jit
def _tc_finish(x, summed, W, b2):
    return pl.pallas_call(
        _tc_body,
        grid=(B // BLK,),
        in_specs=[
            pl.BlockSpec((BLK, H), lambda i: (i, 0)),
            pl.BlockSpec((BLK, D), lambda i: (i, 0)),
            pl.BlockSpec((D, D), lambda i: (0, 0)),
            pl.BlockSpec((1, D), lambda i: (0, 0)),
        ],
        out_specs=pl.BlockSpec((BLK, D), lambda i: (i, 0)),
        out_shape=jax.ShapeDtypeStruct((B, D), jnp.float32),
    )(x, summed, W, b2)


def kernel(x, lengths, emb, W, b):
    x = jnp.asarray(x, jnp.int32)
    summed = _sc_sum(x, emb)
    return _tc_finish(x, summed, W, b.reshape(1, D))
